# baseline (speedup 1.0000x reference)
import numpy as np
import jax
import jax.numpy as jnp
from jax.experimental import pallas as pl
from jax.experimental.pallas import tpu as pltpu
from jax.sharding import Mesh, PartitionSpec as P

_N_CODES = 8

_DQ_ROWS = 8192
_BM = 256


def _butterfly_matrix(g: int) -> np.ndarray:
    x = np.eye(g, dtype=np.float64)
    for _ in range(int(np.log2(g))):
        x = x.reshape(x.shape[:-1] + (2, g // 2))
        a, b = x[..., 0, :], x[..., 1, :]
        x = np.concatenate([a + b, a - b], axis=-1)
    return x / np.sqrt(g)


def _dq_body(idx_ref, nrm_ref, cent_ref, s_ref, w_ref):
    idx = idx_ref[...]
    v = jnp.full(idx.shape, cent_ref[0], dtype=jnp.float32)
    for c in range(1, _N_CODES):
        v = jnp.where(idx == c, cent_ref[c], v)
    v = v * nrm_ref[...]
    w = jax.lax.dot(v.astype(jnp.bfloat16), s_ref[...],
                    preferred_element_type=jnp.float32)
    w_ref[...] = w.astype(jnp.bfloat16)


def _mm_body(x_ref, w_ref, b_ref, o_ref):
    xb = x_ref[...].astype(jnp.bfloat16)
    acc = jax.lax.dot_general(xb, w_ref[...], (((1,), (1,)), ((), ())),
                              preferred_element_type=jnp.float32)
    o_ref[...] = acc + b_ref[...]


def _impl(x2, packed_weight, nrm2, centroids, smat, bias2):
    m_rows, in_dim = x2.shape
    n_rows, g = packed_weight.shape
    out_dim = bias2.shape[-1]

    dq_rows = min(_DQ_ROWS, n_rows)
    w_big = pl.pallas_call(
        _dq_body,
        grid=(n_rows // dq_rows,),
        in_specs=[
            pl.BlockSpec((dq_rows, g), lambda i: (i, 0)),
            pl.BlockSpec((dq_rows, 1), lambda i: (i, 0)),
            pl.BlockSpec(memory_space=pltpu.SMEM),
            pl.BlockSpec((g, g), lambda i: (0, 0)),
        ],
        out_specs=pl.BlockSpec((dq_rows, g), lambda i: (i, 0)),
        out_shape=jax.ShapeDtypeStruct((n_rows, g), jnp.bfloat16),
        compiler_params=pltpu.CompilerParams(
            dimension_semantics=("arbitrary",)),
    )(packed_weight, nrm2, centroids, smat)

    w = w_big.reshape(out_dim, in_dim)

    bm = min(_BM, m_rows)
    out2 = pl.pallas_call(
        _mm_body,
        grid=(m_rows // bm,),
        in_specs=[
            pl.BlockSpec((bm, in_dim), lambda m: (m, 0)),
            pl.BlockSpec((out_dim, in_dim), lambda m: (0, 0)),
            pl.BlockSpec((1, out_dim), lambda m: (0, 0)),
        ],
        out_specs=pl.BlockSpec((bm, out_dim), lambda m: (m, 0)),
        out_shape=jax.ShapeDtypeStruct((m_rows, out_dim), jnp.float32),
        compiler_params=pltpu.CompilerParams(
            dimension_semantics=("arbitrary",),
            vmem_limit_bytes=57 * 1024 * 1024),
    )(x2, w, bias2)
    return out2


def kernel(x, packed_weight, norms, signs1, signs2, centroids, bias):
    b, s, in_dim = x.shape
    out_dim, n_groups = norms.shape
    g = packed_weight.shape[-1]
    n_rows = out_dim * n_groups

    bmat = jnp.asarray(_butterfly_matrix(g), dtype=jnp.float32)
    smat = (signs2[:, None] * bmat * signs1[None, :]).astype(jnp.bfloat16)
    nrm2 = norms.reshape(n_rows, 1)
    x2 = x.reshape(b * s, in_dim)
    bias2 = bias.reshape(1, out_dim)

    devs = jax.devices()
    n_dev = 2 if (len(devs) >= 2 and (b * s) % (2 * _BM) == 0) else 1
    if n_dev == 2:
        mesh = Mesh(np.array(devs[:2]), ("d",))
        out2 = jax.shard_map(
            _impl,
            mesh=mesh,
            in_specs=(P("d", None), P(None, None), P(None, None),
                      P(None), P(None, None), P(None, None)),
            out_specs=P("d", None),
            check_vma=False,
        )(x2, packed_weight, nrm2, centroids, smat, bias2)
    else:
        out2 = _impl(x2, packed_weight, nrm2, centroids, smat, bias2)

    return out2.reshape(b, s, out_dim)


# speedup vs baseline: 1.4219x; 1.4219x over previous
import numpy as np
import jax
import jax.numpy as jnp
from jax.experimental import pallas as pl
from jax.experimental.pallas import tpu as pltpu
_N_CODES = 8

_DQ_ROWS = 8192
_BM = 256


def _butterfly_matrix(g: int) -> np.ndarray:
    x = np.eye(g, dtype=np.float64)
    for _ in range(int(np.log2(g))):
        x = x.reshape(x.shape[:-1] + (2, g // 2))
        a, b = x[..., 0, :], x[..., 1, :]
        x = np.concatenate([a + b, a - b], axis=-1)
    return x / np.sqrt(g)


def _dq_body(idx_ref, nrm_ref, cent_ref, s_ref, w_ref):
    idx = idx_ref[...]
    v = jnp.full(idx.shape, cent_ref[0], dtype=jnp.float32)
    for c in range(1, _N_CODES):
        v = jnp.where(idx == c, cent_ref[c], v)
    v = v * nrm_ref[...]
    w = jax.lax.dot(v.astype(jnp.bfloat16), s_ref[...],
                    preferred_element_type=jnp.float32)
    w_ref[...] = w.astype(jnp.bfloat16)


def _mm_body(x_ref, w_ref, b_ref, o_ref):
    xb = x_ref[...].astype(jnp.bfloat16)
    acc = jax.lax.dot_general(xb, w_ref[...], (((1,), (0,)), ((), ())),
                              preferred_element_type=jnp.float32)
    o_ref[...] = acc + b_ref[...]


def _impl(x2, packed_weight, nrm2, centroids, smat, bias2):
    m_rows, in_dim = x2.shape
    n_rows, g = packed_weight.shape
    out_dim = bias2.shape[-1]

    dq_rows = min(_DQ_ROWS, n_rows)
    w_big = pl.pallas_call(
        _dq_body,
        grid=(n_rows // dq_rows,),
        in_specs=[
            pl.BlockSpec((dq_rows, g), lambda i: (i, 0)),
            pl.BlockSpec((dq_rows, 1), lambda i: (i, 0)),
            pl.BlockSpec(memory_space=pltpu.SMEM),
            pl.BlockSpec((g, g), lambda i: (0, 0)),
        ],
        out_specs=pl.BlockSpec((dq_rows, g), lambda i: (i, 0)),
        out_shape=jax.ShapeDtypeStruct((n_rows, g), jnp.bfloat16),
        compiler_params=pltpu.CompilerParams(
            dimension_semantics=("arbitrary",)),
    )(packed_weight, nrm2, centroids, smat)

    n_groups = in_dim // g
    wt = (w_big.reshape(out_dim, n_groups, g)
          .transpose(1, 2, 0).reshape(in_dim, out_dim))

    bm = min(_BM, m_rows)
    out2 = pl.pallas_call(
        _mm_body,
        grid=(m_rows // bm,),
        in_specs=[
            pl.BlockSpec((bm, in_dim), lambda m: (m, 0)),
            pl.BlockSpec((in_dim, out_dim), lambda m: (0, 0)),
            pl.BlockSpec((1, out_dim), lambda m: (0, 0)),
        ],
        out_specs=pl.BlockSpec((bm, out_dim), lambda m: (m, 0)),
        out_shape=jax.ShapeDtypeStruct((m_rows, out_dim), jnp.float32),
        compiler_params=pltpu.CompilerParams(
            dimension_semantics=("arbitrary",),
            vmem_limit_bytes=57 * 1024 * 1024),
    )(x2, wt, bias2)
    return out2


def kernel(x, packed_weight, norms, signs1, signs2, centroids, bias):
    b, s, in_dim = x.shape
    out_dim, n_groups = norms.shape
    g = packed_weight.shape[-1]
    n_rows = out_dim * n_groups

    bmat = jnp.asarray(_butterfly_matrix(g), dtype=jnp.float32)
    smat = (signs2[:, None] * bmat * signs1[None, :]).astype(jnp.bfloat16)
    nrm2 = norms.reshape(n_rows, 1)
    x2 = x.reshape(b * s, in_dim)
    bias2 = bias.reshape(1, out_dim)

    out2 = _impl(x2, packed_weight, nrm2, centroids, smat, bias2)
    return out2.reshape(b, s, out_dim)


# speedup vs baseline: 1.5901x; 1.1183x over previous
import numpy as np
import jax
import jax.numpy as jnp
from jax.experimental import pallas as pl
from jax.experimental.pallas import tpu as pltpu
_N_CODES = 8

_DQ_ROWS = 8192
_BM = 256


def _butterfly_matrix(g: int) -> np.ndarray:
    x = np.eye(g, dtype=np.float64)
    for _ in range(int(np.log2(g))):
        x = x.reshape(x.shape[:-1] + (2, g // 2))
        a, b = x[..., 0, :], x[..., 1, :]
        x = np.concatenate([a + b, a - b], axis=-1)
    return x / np.sqrt(g)


def _dq_body(idx_ref, nrm_ref, cent_ref, s_ref, w_ref):
    idx = idx_ref[...]
    rows, g = idx.shape
    v = jnp.full(idx.shape, cent_ref[0], dtype=jnp.float32)
    for c in range(1, _N_CODES):
        v = jnp.where(idx == c, cent_ref[c], v)
    n_o, n_g = nrm_ref.shape
    v = (v.reshape(n_o, n_g, g) * nrm_ref[...][:, :, None]).reshape(rows, g)
    w = jax.lax.dot(v.astype(jnp.bfloat16), s_ref[...],
                    preferred_element_type=jnp.float32)
    w_ref[...] = w.astype(jnp.bfloat16)


def _mm_body(x_ref, w_ref, b_ref, o_ref):
    xb = x_ref[...].astype(jnp.bfloat16)
    acc = jax.lax.dot_general(xb, w_ref[...], (((1,), (0,)), ((), ())),
                              preferred_element_type=jnp.float32)
    o_ref[...] = acc + b_ref[...]


def _impl(x2, packed_weight, norms, centroids, smat, bias2):
    m_rows, in_dim = x2.shape
    n_rows, g = packed_weight.shape
    out_dim, n_groups = norms.shape

    dq_rows = min(_DQ_ROWS, n_rows)
    w_big = pl.pallas_call(
        _dq_body,
        grid=(n_rows // dq_rows,),
        in_specs=[
            pl.BlockSpec((dq_rows, g), lambda i: (i, 0)),
            pl.BlockSpec((dq_rows // n_groups, n_groups), lambda i: (i, 0)),
            pl.BlockSpec(memory_space=pltpu.SMEM),
            pl.BlockSpec((g, g), lambda i: (0, 0)),
        ],
        out_specs=pl.BlockSpec((dq_rows, g), lambda i: (i, 0)),
        out_shape=jax.ShapeDtypeStruct((n_rows, g), jnp.bfloat16),
        compiler_params=pltpu.CompilerParams(
            dimension_semantics=("arbitrary",)),
    )(packed_weight, norms, centroids, smat)

    n_groups = in_dim // g
    wt = (w_big.reshape(out_dim, n_groups, g)
          .transpose(1, 2, 0).reshape(in_dim, out_dim))

    bm = min(_BM, m_rows)
    out2 = pl.pallas_call(
        _mm_body,
        grid=(m_rows // bm,),
        in_specs=[
            pl.BlockSpec((bm, in_dim), lambda m: (m, 0)),
            pl.BlockSpec((in_dim, out_dim), lambda m: (0, 0)),
            pl.BlockSpec((1, out_dim), lambda m: (0, 0)),
        ],
        out_specs=pl.BlockSpec((bm, out_dim), lambda m: (m, 0)),
        out_shape=jax.ShapeDtypeStruct((m_rows, out_dim), jnp.float32),
        compiler_params=pltpu.CompilerParams(
            dimension_semantics=("arbitrary",),
            vmem_limit_bytes=57 * 1024 * 1024),
    )(x2, wt, bias2)
    return out2


def kernel(x, packed_weight, norms, signs1, signs2, centroids, bias):
    b, s, in_dim = x.shape
    out_dim, n_groups = norms.shape
    g = packed_weight.shape[-1]
    n_rows = out_dim * n_groups

    bmat = jnp.asarray(_butterfly_matrix(g), dtype=jnp.float32)
    smat = (signs2[:, None] * bmat * signs1[None, :]).astype(jnp.bfloat16)
    x2 = x.reshape(b * s, in_dim)
    bias2 = bias.reshape(1, out_dim)

    out2 = _impl(x2, packed_weight, norms, centroids, smat, bias2)
    return out2.reshape(b, s, out_dim)


# speedup vs baseline: 1.5919x; 1.0011x over previous
import numpy as np
import jax
import jax.numpy as jnp
from jax.experimental import pallas as pl
from jax.experimental.pallas import tpu as pltpu
_N_CODES = 8

_DQ_ROWS = 8192
_BM = 256


def _butterfly_matrix(g: int) -> np.ndarray:
    x = np.eye(g, dtype=np.float64)
    for _ in range(int(np.log2(g))):
        x = x.reshape(x.shape[:-1] + (2, g // 2))
        a, b = x[..., 0, :], x[..., 1, :]
        x = np.concatenate([a + b, a - b], axis=-1)
    return x / np.sqrt(g)


def _dq_body(idx_ref, nrm_ref, cent_ref, s_ref, w_ref):
    idx = idx_ref[...]
    rows, g = idx.shape
    b0 = (idx & 1) == 1
    b1 = (idx & 2) == 2
    b2 = (idx & 4) == 4
    m0 = jnp.where(b0, cent_ref[1], cent_ref[0])
    m1 = jnp.where(b0, cent_ref[3], cent_ref[2])
    m2 = jnp.where(b0, cent_ref[5], cent_ref[4])
    m3 = jnp.where(b0, cent_ref[7], cent_ref[6])
    n0 = jnp.where(b1, m1, m0)
    n1 = jnp.where(b1, m3, m2)
    v = jnp.where(b2, n1, n0)
    n_o, n_g = nrm_ref.shape
    v = (v.reshape(n_o, n_g, g) * nrm_ref[...][:, :, None]).reshape(rows, g)
    w = jax.lax.dot(v.astype(jnp.bfloat16), s_ref[...],
                    preferred_element_type=jnp.float32)
    w_ref[...] = w.astype(jnp.bfloat16)


def _mm_body(x_ref, w_ref, b_ref, o_ref):
    xb = x_ref[...].astype(jnp.bfloat16)
    acc = jax.lax.dot_general(xb, w_ref[...], (((1,), (0,)), ((), ())),
                              preferred_element_type=jnp.float32)
    o_ref[...] = acc + b_ref[...]


def _impl(x2, packed_weight, norms, centroids, smat, bias2):
    m_rows, in_dim = x2.shape
    n_rows, g = packed_weight.shape
    out_dim, n_groups = norms.shape

    dq_rows = min(_DQ_ROWS, n_rows)
    w_big = pl.pallas_call(
        _dq_body,
        grid=(n_rows // dq_rows,),
        in_specs=[
            pl.BlockSpec((dq_rows, g), lambda i: (i, 0)),
            pl.BlockSpec((dq_rows // n_groups, n_groups), lambda i: (i, 0)),
            pl.BlockSpec(memory_space=pltpu.SMEM),
            pl.BlockSpec((g, g), lambda i: (0, 0)),
        ],
        out_specs=pl.BlockSpec((dq_rows, g), lambda i: (i, 0)),
        out_shape=jax.ShapeDtypeStruct((n_rows, g), jnp.bfloat16),
        compiler_params=pltpu.CompilerParams(
            dimension_semantics=("arbitrary",)),
    )(packed_weight, norms, centroids, smat)

    n_groups = in_dim // g
    wt = (w_big.reshape(out_dim, n_groups, g)
          .transpose(1, 2, 0).reshape(in_dim, out_dim))

    bm = min(_BM, m_rows)
    out2 = pl.pallas_call(
        _mm_body,
        grid=(m_rows // bm,),
        in_specs=[
            pl.BlockSpec((bm, in_dim), lambda m: (m, 0)),
            pl.BlockSpec((in_dim, out_dim), lambda m: (0, 0)),
            pl.BlockSpec((1, out_dim), lambda m: (0, 0)),
        ],
        out_specs=pl.BlockSpec((bm, out_dim), lambda m: (m, 0)),
        out_shape=jax.ShapeDtypeStruct((m_rows, out_dim), jnp.float32),
        compiler_params=pltpu.CompilerParams(
            dimension_semantics=("arbitrary",),
            vmem_limit_bytes=57 * 1024 * 1024),
    )(x2, wt, bias2)
    return out2


def kernel(x, packed_weight, norms, signs1, signs2, centroids, bias):
    b, s, in_dim = x.shape
    out_dim, n_groups = norms.shape
    g = packed_weight.shape[-1]
    n_rows = out_dim * n_groups

    bmat = jnp.asarray(_butterfly_matrix(g), dtype=jnp.float32)
    smat = (signs2[:, None] * bmat * signs1[None, :]).astype(jnp.bfloat16)
    x2 = x.reshape(b * s, in_dim)
    bias2 = bias.reshape(1, out_dim)

    out2 = _impl(x2, packed_weight, norms, centroids, smat, bias2)
    return out2.reshape(b, s, out_dim)
